# Initial kernel scaffold; baseline (speedup 1.0000x reference)
import jax
import jax.numpy as jnp
from jax.experimental import pallas as pl


def kernel(x, edge_index, batch, Wl0, bl0, Wr0, Wres0, bres0, Wl1, bl1, Wr1, Wres1, bres1, Wout, bout):
    raise NotImplementedError("write your pallas kernel here")


# speedup vs baseline: 2.7845x; 2.7845x over previous
import functools
import math

import jax
import jax.numpy as jnp
from jax import lax
from jax.experimental import pallas as pl
from jax.experimental.pallas import tpu as pltpu
from jax.experimental.pallas import tpu_sc as plsc

NC = 2
NS = 16
WE = 128


def _ceil_to(a, m):
    return (a + m - 1) // m * m


def _sc_segsum(vals_flat, src_idx, dst_idx, zrows, zrow1, n_acc, ncb,
               with_cnt):
    kw = src_idx.shape[2]
    rt = n_acc // NS
    pp = ncb // NC

    out_type = [jax.ShapeDtypeStruct((ncb, n_acc, 128), jnp.float32)]
    if with_cnt:
        out_type.append(jax.ShapeDtypeStruct((n_acc,), jnp.float32))

    mesh = plsc.VectorSubcoreMesh(core_axis_name="c", subcore_axis_name="s",
                                  num_cores=NC, num_subcores=NS)

    def body(vals_hbm, src_hbm, dst_hbm, zr_hbm, z1_hbm, out_hbm, *rest):
        if with_cnt:
            cnt_hbm, srcv, dstv, rows, ones_v, accum, cnt_acc, sem = rest
        else:
            srcv, dstv, rows, ones_v, accum, sem = rest
        c = lax.axis_index("c")
        s = lax.axis_index("s")

        pltpu.sync_copy(dst_hbm.at[s], dstv)
        if with_cnt:
            for j in range(8):
                ones_v[pl.ds(j * 16, 16)] = jnp.ones((16,), jnp.float32)

        for p in range(pp):
            cb = c * pp + p
            pltpu.sync_copy(zr_hbm, accum.at[pl.ds(s * rt, rt)])
            if with_cnt and p == 0:
                @pl.when(c == 0)
                def _():
                    pltpu.sync_copy(z1_hbm, cnt_acc.at[pl.ds(s * rt, rt)])
            pltpu.sync_copy(src_hbm.at[cb, s], srcv)
            plsc.subcore_barrier()

            def window(k, carry):
                pltpu.async_copy(vals_hbm.at[srcv.at[k]], rows, sem).wait()
                pltpu.sync_copy(rows, accum.at[dstv.at[k]], add=True)
                if with_cnt and p == 0:
                    @pl.when(c == 0)
                    def _():
                        pltpu.sync_copy(ones_v, cnt_acc.at[dstv.at[k]],
                                        add=True)
                return carry

            lax.fori_loop(0, kw, window, 0)
            plsc.subcore_barrier()

            pltpu.sync_copy(accum.at[pl.ds(s * rt, rt)],
                            out_hbm.at[cb, pl.ds(s * rt, rt)])
            if with_cnt and p == 0:
                @pl.when(c == 0)
                def _():
                    pltpu.sync_copy(cnt_acc.at[pl.ds(s * rt, rt)],
                                    cnt_hbm.at[pl.ds(s * rt, rt)])
            if p + 1 < pp:
                plsc.subcore_barrier()

    scratch = [
        pltpu.VMEM((kw, 128), jnp.int32),
        pltpu.VMEM((kw, 128), jnp.int32),
        pltpu.VMEM((WE, 128), jnp.float32),
        pltpu.VMEM((WE,), jnp.float32),
        pltpu.VMEM_SHARED((n_acc, 128), jnp.float32),
    ]
    if with_cnt:
        scratch.append(pltpu.VMEM_SHARED((n_acc,), jnp.float32))
    scratch.append(pltpu.SemaphoreType.DMA)

    fn = pl.kernel(body, out_type=tuple(out_type), mesh=mesh,
                   scratch_types=tuple(scratch))
    return fn(vals_flat, src_idx, dst_idx, zrows, zrow1)


def _tc_layer0(x, agg, cnt2d, wl, wc, b2d, n, bn):
    grid = n // bn

    def body(x_ref, agg_ref, cnt_ref, wl_ref, wc_ref, b_ref, out_ref):
        recip = 1.0 / jnp.maximum(cnt_ref[...], 1.0)
        acc = jnp.dot(agg_ref[0], wl_ref[0],
                      preferred_element_type=jnp.float32)
        acc += jnp.dot(agg_ref[1], wl_ref[1],
                       preferred_element_type=jnp.float32)
        acc *= recip
        acc += jnp.dot(x_ref[...], wc_ref[...],
                       preferred_element_type=jnp.float32)
        acc += b_ref[...]
        out_ref[...] = jnp.maximum(acc, 0.0)

    return pl.pallas_call(
        body,
        grid=(grid,),
        in_specs=[
            pl.BlockSpec((bn, 256), lambda i: (i, 0)),
            pl.BlockSpec((2, bn, 128), lambda i: (0, i, 0)),
            pl.BlockSpec((bn, 1), lambda i: (i, 0)),
            pl.BlockSpec((2, 128, 512), lambda i: (0, 0, 0)),
            pl.BlockSpec((256, 512), lambda i: (0, 0)),
            pl.BlockSpec((1, 512), lambda i: (0, 0)),
        ],
        out_specs=pl.BlockSpec((bn, 512), lambda i: (i, 0)),
        out_shape=jax.ShapeDtypeStruct((n, 512), jnp.float32),
        compiler_params=pltpu.CompilerParams(
            dimension_semantics=("parallel",)),
    )(x, agg, cnt2d, wl, wc, b2d)


def _tc_layer1(h, agg, cnt2d, wl, wc, b2d, wout, n, bn):
    grid = n // bn

    def body(h_ref, agg_ref, cnt_ref, wl_ref, wc_ref, b_ref, wout_ref,
             z_ref):
        recip = 1.0 / jnp.maximum(cnt_ref[...], 1.0)
        acc = jnp.dot(agg_ref[0], wl_ref[0],
                      preferred_element_type=jnp.float32)
        for q in range(1, 4):
            acc += jnp.dot(agg_ref[q], wl_ref[q],
                           preferred_element_type=jnp.float32)
        acc *= recip
        acc += jnp.dot(h_ref[...], wc_ref[...],
                       preferred_element_type=jnp.float32)
        acc += b_ref[...]
        z_ref[...] = jnp.dot(acc, wout_ref[...],
                             preferred_element_type=jnp.float32)

    return pl.pallas_call(
        body,
        grid=(grid,),
        in_specs=[
            pl.BlockSpec((bn, 512), lambda i: (i, 0)),
            pl.BlockSpec((4, bn, 128), lambda i: (0, i, 0)),
            pl.BlockSpec((bn, 1), lambda i: (i, 0)),
            pl.BlockSpec((4, 128, 512), lambda i: (0, 0, 0)),
            pl.BlockSpec((512, 512), lambda i: (0, 0)),
            pl.BlockSpec((1, 512), lambda i: (0, 0)),
            pl.BlockSpec((512, 1), lambda i: (0, 0)),
        ],
        out_specs=pl.BlockSpec((bn, 1), lambda i: (i, 0)),
        out_shape=jax.ShapeDtypeStruct((n, 1), jnp.float32),
        compiler_params=pltpu.CompilerParams(
            dimension_semantics=("parallel",)),
    )(h, agg, cnt2d, wl, wc, b2d, wout)


def _tc_pool(z, batch2d, bout, n, nb):
    def body(z_ref, bt_ref, bout_ref, out_ref):
        zv = z_ref[...]
        bt = bt_ref[...]
        ids = lax.broadcasted_iota(jnp.int32, (1, nb), 1)
        onehot = (bt == ids).astype(jnp.float32)
        dn = (((0,), (0,)), ((), ()))
        sums = lax.dot_general(onehot, zv, dn,
                               preferred_element_type=jnp.float32)
        cnts = jnp.sum(onehot, axis=0, keepdims=True).T
        out_ref[...] = sums / jnp.maximum(cnts, 1.0) + bout_ref[...]

    return pl.pallas_call(
        body,
        grid=(1,),
        in_specs=[
            pl.BlockSpec((n, 1), lambda i: (0, 0)),
            pl.BlockSpec((n, 1), lambda i: (0, 0)),
            pl.BlockSpec((1, 1), lambda i: (0, 0)),
        ],
        out_specs=pl.BlockSpec((nb, 1), lambda i: (0, 0)),
        out_shape=jax.ShapeDtypeStruct((nb, 1), jnp.float32),
    )(z, batch2d, bout)


def kernel(x, edge_index, batch, Wl0, bl0, Wr0, Wres0, bres0,
           Wl1, bl1, Wr1, Wres1, bres1, Wout, bout):
    n, d = x.shape
    h = Wl0.shape[1]
    e = edge_index.shape[1]
    nb = 8

    src = edge_index[0]
    dst = edge_index[1]

    n_acc = _ceil_to(n, NS * 16)
    trash = n_acc - n
    kw = math.ceil(math.ceil(e / NS) / WE)
    kw = kw + (kw % 2)
    e_pad = NS * kw * WE
    npad = e_pad - e

    pad_src = jnp.zeros((npad,), jnp.int32)
    pad_dst = (n + jnp.arange(npad, dtype=jnp.int32) % max(trash, 1))
    src_p = jnp.concatenate([src, pad_src])
    dst_p = jnp.concatenate([dst, pad_dst.astype(jnp.int32)])

    ncb0 = d // 128
    ncb1 = h // 128
    cb0 = jnp.arange(ncb0, dtype=jnp.int32)[:, None]
    cb1 = jnp.arange(ncb1, dtype=jnp.int32)[:, None]
    src_idx0 = (src_p[None, :] * ncb0 + cb0).reshape(ncb0, NS, kw, WE)
    src_idx1 = (src_p[None, :] * ncb1 + cb1).reshape(ncb1, NS, kw, WE)
    dst_idx = dst_p.reshape(NS, kw, WE)

    rt = n_acc // NS
    zrows = jnp.zeros((rt, 128), jnp.float32)
    zrow1 = jnp.zeros((rt,), jnp.float32)

    agg0, cnt = _sc_segsum(x.reshape(n * ncb0, 128), src_idx0, dst_idx,
                           zrows, zrow1, n_acc, ncb0, True)
    b0 = (bl0 + bres0).reshape(1, h)
    cnt2d = cnt.reshape(n_acc, 1)
    h0 = _tc_layer0(x, agg0, cnt2d, Wl0.reshape(2, 128, h), Wr0 + Wres0,
                    b0, n, 1000)

    (agg1,) = _sc_segsum(h0.reshape(n * ncb1, 128), src_idx1, dst_idx,
                         zrows, zrow1, n_acc, ncb1, False)
    b1 = (bl1 + bres1).reshape(1, h)
    z = _tc_layer1(h0, agg1, cnt2d, Wl1.reshape(4, 128, h), Wr1 + Wres1,
                   b1, Wout, n, 1000)

    return _tc_pool(z, batch.reshape(n, 1), bout.reshape(1, 1), n, nb)


# speedup vs baseline: 3.0236x; 1.0859x over previous
import functools
import math

import jax
import jax.numpy as jnp
from jax import lax
from jax.experimental import pallas as pl
from jax.experimental.pallas import tpu as pltpu
from jax.experimental.pallas import tpu_sc as plsc

NC = 2
NS = 16
WE = 128
CH = 40


def _ceil_to(a, m):
    return (a + m - 1) // m * m


def _sc_segsum(vals_flat, src_idx, dst_idx, zrows, zrow1, n_acc, ncb,
               with_cnt):
    kw = src_idx.shape[1]
    rt = n_acc // NS
    pp = ncb // NC

    out_type = [jax.ShapeDtypeStruct((ncb, n_acc, 128), jnp.float32)]
    if with_cnt:
        out_type.append(jax.ShapeDtypeStruct((n_acc,), jnp.float32))

    mesh = plsc.VectorSubcoreMesh(core_axis_name="c", subcore_axis_name="s",
                                  num_cores=NC, num_subcores=NS)

    def body(vals_hbm, src_hbm, dst_hbm, zr_hbm, z1_hbm, out_hbm, *rest):
        if with_cnt:
            (cnt_hbm, srcv, dstv, rows_a, rows_b, ones_v, accum,
             cnt_acc, sem_a, sem_b) = rest
        else:
            (srcv, dstv, rows_a, rows_b, ones_v, accum, sem_a,
             sem_b) = rest
        c = lax.axis_index("c")
        s = lax.axis_index("s")

        if with_cnt:
            for j in range(8):
                ones_v[pl.ds(j * 16, 16)] = jnp.ones((16,), jnp.float32)

        for p in range(pp):
            cb = c * pp + p
            pltpu.sync_copy(zr_hbm, accum.at[pl.ds(s * rt, rt)])
            if with_cnt and p == 0:
                @pl.when(c == 0)
                def _():
                    pltpu.sync_copy(z1_hbm, cnt_acc.at[pl.ds(s * rt, rt)])
            plsc.subcore_barrier()

            def gissue(k, buf, sem):
                pltpu.async_copy(vals_hbm.at[srcv.at[k]], buf, sem)

            def gwait(k, buf, sem):
                pltpu.make_async_copy(vals_hbm.at[srcv.at[k]], buf,
                                      sem).wait()

            def scat(k, buf):
                pltpu.sync_copy(buf, accum.at[dstv.at[k]], add=True)
                if with_cnt and p == 0:
                    @pl.when(c == 0)
                    def _():
                        pltpu.sync_copy(ones_v, cnt_acc.at[dstv.at[k]],
                                        add=True)

            for ch in range(kw // CH):
                pltpu.sync_copy(src_hbm.at[s, pl.ds(ch * CH, CH)], srcv)
                pltpu.sync_copy(dst_hbm.at[s, pl.ds(ch * CH, CH)], dstv)

                def sscale(k, carry):
                    for j in range(8):
                        sl = pl.ds(j * 16, 16)
                        srcv[k, sl] = srcv[k, sl] * ncb + cb
                    return carry

                lax.fori_loop(0, CH, sscale, 0)

                gissue(0, rows_a, sem_a)

                def pair(i, carry):
                    k = 2 * i
                    gwait(k, rows_a, sem_a)
                    gissue(k + 1, rows_b, sem_b)
                    scat(k, rows_a)
                    gwait(k + 1, rows_b, sem_b)

                    @pl.when(k + 2 < CH)
                    def _():
                        gissue(k + 2, rows_a, sem_a)

                    scat(k + 1, rows_b)
                    return carry

                lax.fori_loop(0, CH // 2, pair, 0)
            plsc.subcore_barrier()

            pltpu.sync_copy(accum.at[pl.ds(s * rt, rt)],
                            out_hbm.at[cb, pl.ds(s * rt, rt)])
            if with_cnt and p == 0:
                @pl.when(c == 0)
                def _():
                    pltpu.sync_copy(cnt_acc.at[pl.ds(s * rt, rt)],
                                    cnt_hbm.at[pl.ds(s * rt, rt)])
            if p + 1 < pp:
                plsc.subcore_barrier()

    scratch = [
        pltpu.VMEM((CH, 128), jnp.int32),
        pltpu.VMEM((CH, 128), jnp.int32),
        pltpu.VMEM((WE, 128), jnp.float32),
        pltpu.VMEM((WE, 128), jnp.float32),
        pltpu.VMEM((WE,), jnp.float32),
        pltpu.VMEM_SHARED((n_acc, 128), jnp.float32),
    ]
    if with_cnt:
        scratch.append(pltpu.VMEM_SHARED((n_acc,), jnp.float32))
    scratch.append(pltpu.SemaphoreType.DMA)
    scratch.append(pltpu.SemaphoreType.DMA)

    fn = pl.kernel(body, out_type=tuple(out_type), mesh=mesh,
                   scratch_types=tuple(scratch))
    return fn(vals_flat, src_idx, dst_idx, zrows, zrow1)


def _tc_layer0(x, agg, cnt2d, wl, wc, b2d, n, bn):
    grid = n // bn

    def body(x_ref, agg_ref, cnt_ref, wl_ref, wc_ref, b_ref, out_ref):
        recip = 1.0 / jnp.maximum(cnt_ref[...], 1.0)
        acc = jnp.dot(agg_ref[0], wl_ref[0],
                      preferred_element_type=jnp.float32)
        acc += jnp.dot(agg_ref[1], wl_ref[1],
                       preferred_element_type=jnp.float32)
        acc *= recip
        acc += jnp.dot(x_ref[...], wc_ref[...],
                       preferred_element_type=jnp.float32)
        acc += b_ref[...]
        out_ref[...] = jnp.maximum(acc, 0.0)

    return pl.pallas_call(
        body,
        grid=(grid,),
        in_specs=[
            pl.BlockSpec((bn, 256), lambda i: (i, 0)),
            pl.BlockSpec((2, bn, 128), lambda i: (0, i, 0)),
            pl.BlockSpec((bn, 1), lambda i: (i, 0)),
            pl.BlockSpec((2, 128, 512), lambda i: (0, 0, 0)),
            pl.BlockSpec((256, 512), lambda i: (0, 0)),
            pl.BlockSpec((1, 512), lambda i: (0, 0)),
        ],
        out_specs=pl.BlockSpec((bn, 512), lambda i: (i, 0)),
        out_shape=jax.ShapeDtypeStruct((n, 512), jnp.float32),
        compiler_params=pltpu.CompilerParams(
            dimension_semantics=("parallel",)),
    )(x, agg, cnt2d, wl, wc, b2d)


def _tc_layer1(h, agg, cnt2d, wl, wc, b2d, wout, n, bn):
    grid = n // bn

    def body(h_ref, agg_ref, cnt_ref, wl_ref, wc_ref, b_ref, wout_ref,
             z_ref):
        recip = 1.0 / jnp.maximum(cnt_ref[...], 1.0)
        acc = jnp.dot(agg_ref[0], wl_ref[0],
                      preferred_element_type=jnp.float32)
        for q in range(1, 4):
            acc += jnp.dot(agg_ref[q], wl_ref[q],
                           preferred_element_type=jnp.float32)
        acc *= recip
        acc += jnp.dot(h_ref[...], wc_ref[...],
                       preferred_element_type=jnp.float32)
        acc += b_ref[...]
        z_ref[...] = jnp.dot(acc, wout_ref[...],
                             preferred_element_type=jnp.float32)

    return pl.pallas_call(
        body,
        grid=(grid,),
        in_specs=[
            pl.BlockSpec((bn, 512), lambda i: (i, 0)),
            pl.BlockSpec((4, bn, 128), lambda i: (0, i, 0)),
            pl.BlockSpec((bn, 1), lambda i: (i, 0)),
            pl.BlockSpec((4, 128, 512), lambda i: (0, 0, 0)),
            pl.BlockSpec((512, 512), lambda i: (0, 0)),
            pl.BlockSpec((1, 512), lambda i: (0, 0)),
            pl.BlockSpec((512, 1), lambda i: (0, 0)),
        ],
        out_specs=pl.BlockSpec((bn, 1), lambda i: (i, 0)),
        out_shape=jax.ShapeDtypeStruct((n, 1), jnp.float32),
        compiler_params=pltpu.CompilerParams(
            dimension_semantics=("parallel",)),
    )(h, agg, cnt2d, wl, wc, b2d, wout)


def _tc_pool(z, batch2d, bout, n, nb):
    def body(z_ref, bt_ref, bout_ref, out_ref):
        zv = z_ref[...]
        bt = bt_ref[...]
        ids = lax.broadcasted_iota(jnp.int32, (1, nb), 1)
        onehot = (bt == ids).astype(jnp.float32)
        dn = (((0,), (0,)), ((), ()))
        sums = lax.dot_general(onehot, zv, dn,
                               preferred_element_type=jnp.float32)
        cnts = jnp.sum(onehot, axis=0, keepdims=True).T
        out_ref[...] = sums / jnp.maximum(cnts, 1.0) + bout_ref[...]

    return pl.pallas_call(
        body,
        grid=(1,),
        in_specs=[
            pl.BlockSpec((n, 1), lambda i: (0, 0)),
            pl.BlockSpec((n, 1), lambda i: (0, 0)),
            pl.BlockSpec((1, 1), lambda i: (0, 0)),
        ],
        out_specs=pl.BlockSpec((nb, 1), lambda i: (0, 0)),
        out_shape=jax.ShapeDtypeStruct((nb, 1), jnp.float32),
    )(z, batch2d, bout)


def kernel(x, edge_index, batch, Wl0, bl0, Wr0, Wres0, bres0,
           Wl1, bl1, Wr1, Wres1, bres1, Wout, bout):
    n, d = x.shape
    h = Wl0.shape[1]
    e = edge_index.shape[1]
    nb = 8

    src = edge_index[0]
    dst = edge_index[1]

    n_acc = _ceil_to(n, NS * 128)
    trash = n_acc - n
    kw = _ceil_to(math.ceil(math.ceil(e / NS) / WE), CH)
    e_pad = NS * kw * WE
    npad = e_pad - e

    pad_src = jnp.zeros((npad,), jnp.int32)
    pad_dst = (n + jnp.arange(npad, dtype=jnp.int32) % max(trash, 1))
    src_p = jnp.concatenate([src, pad_src])
    dst_p = jnp.concatenate([dst, pad_dst.astype(jnp.int32)])

    ncb0 = d // 128
    ncb1 = h // 128
    src_idx = src_p.reshape(NS, kw, WE)
    dst_idx = dst_p.reshape(NS, kw, WE)

    rt = n_acc // NS
    zrows = jnp.zeros((rt, 128), jnp.float32)
    zrow1 = jnp.zeros((rt,), jnp.float32)

    agg0, cnt = _sc_segsum(x.reshape(n * ncb0, 128), src_idx, dst_idx,
                           zrows, zrow1, n_acc, ncb0, True)
    b0 = (bl0 + bres0).reshape(1, h)
    cnt2d = cnt.reshape(n_acc, 1)
    h0 = _tc_layer0(x, agg0, cnt2d, Wl0.reshape(2, 128, h), Wr0 + Wres0,
                    b0, n, 1000)

    (agg1,) = _sc_segsum(h0.reshape(n * ncb1, 128), src_idx, dst_idx,
                         zrows, zrow1, n_acc, ncb1, False)
    b1 = (bl1 + bres1).reshape(1, h)
    z = _tc_layer1(h0, agg1, cnt2d, Wl1.reshape(4, 128, h), Wr1 + Wres1,
                   b1, Wout, n, 1000)

    return _tc_pool(z, batch.reshape(n, 1), bout.reshape(1, 1), n, nb)
